# Initial kernel scaffold; baseline (speedup 1.0000x reference)
import jax
import jax.numpy as jnp
from jax.experimental import pallas as pl


def kernel(words, word_len, ngrams, ngram_len, ngram_table, word_table, W1, b1, W2, b2):
    raise NotImplementedError("write your pallas kernel here")


# speedup vs baseline: 2.0685x; 2.0685x over previous
import functools

import jax
import jax.numpy as jnp
from jax import lax
from jax.experimental import pallas as pl
from jax.experimental.pallas import tpu as pltpu
from jax.experimental.pallas import tpu_sc as plsc


@functools.cache
def _pooled_sum_kernel(B: int, L: int, V: int, D: int, CB: int):
    info = plsc.get_sparse_core_info()
    NC, NS = info.num_cores, info.num_subcores
    NW = NC * NS
    assert B % (NW * CB) == 0
    PB = B // NW
    n_chunks = PB // CB

    mesh = plsc.VectorSubcoreMesh(core_axis_name="c", subcore_axis_name="s")

    @functools.partial(
        pl.kernel,
        mesh=mesh,
        out_type=jax.ShapeDtypeStruct((B, D), jnp.float32),
        compiler_params=pltpu.CompilerParams(use_tc_tiling_on_sc=False),
        scratch_types=[
            pltpu.VMEM((CB * L,), jnp.int32),
            pltpu.VMEM((CB * L, D), jnp.float32),
            pltpu.VMEM((CB, D), jnp.float32),
            pltpu.SemaphoreType.DMA,
        ],
    )
    def k(table_hbm, idx_hbm, out_hbm, idx_v, rows_v, acc_v, sem):
        wid = lax.axis_index("s") * NC + lax.axis_index("c")

        def chunk_body(c, carry):
            base = wid * PB + c * CB
            pltpu.sync_copy(idx_hbm.at[pl.ds(base * L, CB * L)], idx_v)
            pltpu.async_copy(table_hbm.at[idx_v], rows_v, sem).wait()

            def batch_body(b, carry2):
                def seg_body(j, acc):
                    a0, a1 = acc
                    r = b * L + j
                    a0 = a0 + rows_v[r, pl.ds(0, 16)]
                    a1 = a1 + rows_v[r, pl.ds(16, 16)]
                    return (a0, a1)

                z = jnp.zeros((16,), jnp.float32)
                a0, a1 = lax.fori_loop(0, L, seg_body, (z, z))
                acc_v[b, pl.ds(0, 16)] = a0
                acc_v[b, pl.ds(16, 16)] = a1
                return carry2

            lax.fori_loop(0, CB, batch_body, 0)
            pltpu.sync_copy(acc_v, out_hbm.at[pl.ds(base, CB)])
            return carry

        lax.fori_loop(0, n_chunks, chunk_body, 0)

    return k


def _head_body(s1_ref, s2_ref, nl_ref, wl_ref, w1_ref, b1_ref, w2_ref,
               b2_ref, o_ref):
    x1 = s1_ref[...] / nl_ref[...]
    x2 = s2_ref[...] / wl_ref[...]
    h = jnp.tanh(jnp.concatenate([x1, x2], axis=1))
    u = lax.dot_general(h, w1_ref[...], (((1,), (1,)), ((), ())),
                        preferred_element_type=jnp.float32) + b1_ref[...]
    o_ref[...] = lax.dot_general(u, w2_ref[...], (((1,), (1,)), ((), ())),
                                 preferred_element_type=jnp.float32) + b2_ref[...]


def _head(s1, s2, ngram_len, word_len, W1, b1, W2, b2):
    B, D = s1.shape
    OUTV, OUTD = W2.shape
    BM = 512
    grid = (B // BM,)
    nl = ngram_len.reshape(B, 1)
    wl = word_len.reshape(B, 1)
    return pl.pallas_call(
        _head_body,
        grid=grid,
        in_specs=[
            pl.BlockSpec((BM, D), lambda i: (i, 0)),
            pl.BlockSpec((BM, D), lambda i: (i, 0)),
            pl.BlockSpec((BM, 1), lambda i: (i, 0)),
            pl.BlockSpec((BM, 1), lambda i: (i, 0)),
            pl.BlockSpec((OUTD, 2 * D), lambda i: (0, 0)),
            pl.BlockSpec((1, OUTD), lambda i: (0, 0)),
            pl.BlockSpec((OUTV, OUTD), lambda i: (0, 0)),
            pl.BlockSpec((1, OUTV), lambda i: (0, 0)),
        ],
        out_specs=pl.BlockSpec((BM, OUTV), lambda i: (i, 0)),
        out_shape=jax.ShapeDtypeStruct((B, OUTV), jnp.float32),
    )(s1, s2, nl, wl, W1, b1.reshape(1, OUTD), W2, b2.reshape(1, OUTV))


def kernel(words, word_len, ngrams, ngram_len, ngram_table, word_table,
           W1, b1, W2, b2):
    B, LW = words.shape
    _, LN = ngrams.shape
    WV, WD = word_table.shape
    NV, ND = ngram_table.shape

    ngrams_flat = ngrams.astype(jnp.int32).reshape(-1)
    words_flat = words.astype(jnp.int32).reshape(-1)

    s1 = _pooled_sum_kernel(B, LN, NV, ND, 16)(ngram_table, ngrams_flat)
    s2 = _pooled_sum_kernel(B, LW, WV, WD, 16)(word_table, words_flat)
    return _head(s1, s2, ngram_len, word_len, W1, b1, W2, b2)


# speedup vs baseline: 2.1373x; 1.0333x over previous
import functools

import jax
import jax.numpy as jnp
from jax import lax
from jax.experimental import pallas as pl
from jax.experimental.pallas import tpu as pltpu
from jax.experimental.pallas import tpu_sc as plsc


@functools.cache
def _pooled_sum_kernel(B: int, L: int, V: int, D: int, CB: int):
    info = plsc.get_sparse_core_info()
    NC, NS = info.num_cores, info.num_subcores
    NW = NC * NS
    assert B % (NW * CB) == 0
    PB = B // NW
    n_chunks = PB // CB
    assert n_chunks % 2 == 0
    npairs = n_chunks // 2
    assert L % 2 == 0

    mesh = plsc.VectorSubcoreMesh(core_axis_name="c", subcore_axis_name="s")

    @functools.partial(
        pl.kernel,
        mesh=mesh,
        out_type=jax.ShapeDtypeStruct((B, D), jnp.float32),
        compiler_params=pltpu.CompilerParams(use_tc_tiling_on_sc=False),
        scratch_types=[
            pltpu.VMEM((CB * L,), jnp.int32),
            pltpu.VMEM((CB * L,), jnp.int32),
            pltpu.VMEM((CB * L, D), jnp.float32),
            pltpu.VMEM((CB * L, D), jnp.float32),
            pltpu.VMEM((CB, D), jnp.float32),
            pltpu.VMEM((CB, D), jnp.float32),
            pltpu.SemaphoreType.DMA,
            pltpu.SemaphoreType.DMA,
        ],
    )
    def k(table_hbm, idx_hbm, out_hbm, idx0, idx1, rows0, rows1,
          acc0, acc1, sem0, sem1):
        wid = lax.axis_index("s") * NC + lax.axis_index("c")
        wbase = wid * PB

        def accum_chunk(rows_v, acc_v):
            def batch_body(b, carry2):
                r0 = b * L
                z = jnp.zeros((16,), jnp.float32)
                a0 = a1 = a2 = a3 = z
                for j in range(0, L, 2):
                    a0 = a0 + rows_v[r0 + j, pl.ds(0, 16)]
                    a1 = a1 + rows_v[r0 + j, pl.ds(16, 16)]
                    a2 = a2 + rows_v[r0 + j + 1, pl.ds(0, 16)]
                    a3 = a3 + rows_v[r0 + j + 1, pl.ds(16, 16)]
                acc_v[b, pl.ds(0, 16)] = a0 + a2
                acc_v[b, pl.ds(16, 16)] = a1 + a3
                return carry2

            lax.fori_loop(0, CB, batch_body, 0)

        def stage_and_fire(c, idx_v, rows_v, sem):
            base = wbase + c * CB
            pltpu.sync_copy(idx_hbm.at[pl.ds(base * L, CB * L)], idx_v)
            pltpu.async_copy(table_hbm.at[idx_v], rows_v, sem)

        stage_and_fire(0, idx0, rows0, sem0)

        def pair_body(i, carry):
            c0 = 2 * i
            stage_and_fire(c0 + 1, idx1, rows1, sem1)
            pltpu.make_async_copy(table_hbm.at[idx0], rows0, sem0).wait()
            accum_chunk(rows0, acc0)
            pltpu.sync_copy(acc0, out_hbm.at[pl.ds(wbase + c0 * CB, CB)])

            @pl.when(i + 1 < npairs)
            def _():
                stage_and_fire(c0 + 2, idx0, rows0, sem0)

            pltpu.make_async_copy(table_hbm.at[idx1], rows1, sem1).wait()
            accum_chunk(rows1, acc1)
            pltpu.sync_copy(acc1, out_hbm.at[pl.ds(wbase + (c0 + 1) * CB, CB)])
            return carry

        lax.fori_loop(0, npairs, pair_body, 0)

    return k


def _head_body(s1_ref, s2_ref, nl_ref, wl_ref, w1_ref, b1_ref, w2_ref,
               b2_ref, o_ref):
    x1 = s1_ref[...] / nl_ref[...]
    x2 = s2_ref[...] / wl_ref[...]
    h = jnp.tanh(jnp.concatenate([x1, x2], axis=1))
    u = lax.dot_general(h, w1_ref[...], (((1,), (1,)), ((), ())),
                        preferred_element_type=jnp.float32) + b1_ref[...]
    o_ref[...] = lax.dot_general(u, w2_ref[...], (((1,), (1,)), ((), ())),
                                 preferred_element_type=jnp.float32) + b2_ref[...]


def _head(s1, s2, ngram_len, word_len, W1, b1, W2, b2):
    B, D = s1.shape
    OUTV, OUTD = W2.shape
    BM = 512
    grid = (B // BM,)
    nl = ngram_len.reshape(B, 1)
    wl = word_len.reshape(B, 1)
    return pl.pallas_call(
        _head_body,
        grid=grid,
        in_specs=[
            pl.BlockSpec((BM, D), lambda i: (i, 0)),
            pl.BlockSpec((BM, D), lambda i: (i, 0)),
            pl.BlockSpec((BM, 1), lambda i: (i, 0)),
            pl.BlockSpec((BM, 1), lambda i: (i, 0)),
            pl.BlockSpec((OUTD, 2 * D), lambda i: (0, 0)),
            pl.BlockSpec((1, OUTD), lambda i: (0, 0)),
            pl.BlockSpec((OUTV, OUTD), lambda i: (0, 0)),
            pl.BlockSpec((1, OUTV), lambda i: (0, 0)),
        ],
        out_specs=pl.BlockSpec((BM, OUTV), lambda i: (i, 0)),
        out_shape=jax.ShapeDtypeStruct((B, OUTV), jnp.float32),
    )(s1, s2, nl, wl, W1, b1.reshape(1, OUTD), W2, b2.reshape(1, OUTV))


def kernel(words, word_len, ngrams, ngram_len, ngram_table, word_table,
           W1, b1, W2, b2):
    B, LW = words.shape
    _, LN = ngrams.shape
    WV, WD = word_table.shape
    NV, ND = ngram_table.shape

    ngrams_flat = ngrams.astype(jnp.int32).reshape(-1)
    words_flat = words.astype(jnp.int32).reshape(-1)

    s1 = _pooled_sum_kernel(B, LN, NV, ND, 32)(ngram_table, ngrams_flat)
    s2 = _pooled_sum_kernel(B, LW, WV, WD, 64)(word_table, words_flat)
    return _head(s1, s2, ngram_len, word_len, W1, b1, W2, b2)


# speedup vs baseline: 2.2903x; 1.0716x over previous
import functools

import jax
import jax.numpy as jnp
from jax import lax
from jax.experimental import pallas as pl
from jax.experimental.pallas import tpu as pltpu
from jax.experimental.pallas import tpu_sc as plsc


@functools.cache
def _pooled_sum_kernel(B: int, L: int, V: int, D: int, CB: int):
    info = plsc.get_sparse_core_info()
    NC, NS = info.num_cores, info.num_subcores
    NW = NC * NS
    assert B % (NW * CB) == 0 and CB % 16 == 0
    PB = B // NW
    n_chunks = PB // CB
    assert n_chunks % 2 == 0
    npairs = n_chunks // 2
    assert L % 2 == 0

    mesh = plsc.VectorSubcoreMesh(core_axis_name="c", subcore_axis_name="s")

    @functools.partial(
        pl.kernel,
        mesh=mesh,
        out_type=jax.ShapeDtypeStruct((B, D), jnp.float32),
        compiler_params=pltpu.CompilerParams(use_tc_tiling_on_sc=False),
        scratch_types=[
            pltpu.VMEM((L, CB), jnp.int32),
            pltpu.VMEM((L, CB), jnp.int32),
            pltpu.VMEM((CB * L,), jnp.int32),
            pltpu.VMEM((CB * L,), jnp.int32),
            pltpu.VMEM((CB * L, D), jnp.float32),
            pltpu.VMEM((CB * L, D), jnp.float32),
            pltpu.VMEM((CB, D), jnp.float32),
            pltpu.VMEM((CB, D), jnp.float32),
            pltpu.SemaphoreType.DMA,
            pltpu.SemaphoreType.DMA,
        ],
    )
    def k(table_hbm, idxt_hbm, out_hbm, st0, st1, idx0, idx1, rows0, rows1,
          acc0, acc1, sem0, sem1):
        wid = lax.axis_index("s") * NC + lax.axis_index("c")
        wbase = wid * PB

        def accum_chunk(rows_v, acc_v):
            def batch_body(b, carry2):
                z = jnp.zeros((16,), jnp.float32)
                a0 = a1 = a2 = a3 = z
                for j in range(0, L, 2):
                    a0 = a0 + rows_v[j * CB + b, pl.ds(0, 16)]
                    a1 = a1 + rows_v[j * CB + b, pl.ds(16, 16)]
                    a2 = a2 + rows_v[(j + 1) * CB + b, pl.ds(0, 16)]
                    a3 = a3 + rows_v[(j + 1) * CB + b, pl.ds(16, 16)]
                acc_v[b, pl.ds(0, 16)] = a0 + a2
                acc_v[b, pl.ds(16, 16)] = a1 + a3
                return carry2

            lax.fori_loop(0, CB, batch_body, 0)

        def stage_and_fire(c, st_v, idx_v, rows_v, sem):
            base = wbase + c * CB
            pltpu.sync_copy(idxt_hbm.at[:, pl.ds(base, CB)], st_v)
            for j in range(L):
                for kk in range(CB // 16):
                    idx_v[pl.ds(j * CB + 16 * kk, 16)] = st_v[j, pl.ds(16 * kk, 16)]
            pltpu.async_copy(table_hbm.at[idx_v], rows_v, sem)

        stage_and_fire(0, st0, idx0, rows0, sem0)

        def pair_body(i, carry):
            c0 = 2 * i
            stage_and_fire(c0 + 1, st1, idx1, rows1, sem1)
            pltpu.make_async_copy(table_hbm.at[idx0], rows0, sem0).wait()
            accum_chunk(rows0, acc0)
            pltpu.sync_copy(acc0, out_hbm.at[pl.ds(wbase + c0 * CB, CB)])

            @pl.when(i + 1 < npairs)
            def _():
                stage_and_fire(c0 + 2, st0, idx0, rows0, sem0)

            pltpu.make_async_copy(table_hbm.at[idx1], rows1, sem1).wait()
            accum_chunk(rows1, acc1)
            pltpu.sync_copy(acc1, out_hbm.at[pl.ds(wbase + (c0 + 1) * CB, CB)])
            return carry

        lax.fori_loop(0, npairs, pair_body, 0)

    return k


def _head_body(s1_ref, s2_ref, nl_ref, wl_ref, w1_ref, b1_ref, w2_ref,
               b2_ref, o_ref):
    x1 = s1_ref[...] / nl_ref[...]
    x2 = s2_ref[...] / wl_ref[...]
    h = jnp.tanh(jnp.concatenate([x1, x2], axis=1))
    u = lax.dot_general(h, w1_ref[...], (((1,), (1,)), ((), ())),
                        preferred_element_type=jnp.float32) + b1_ref[...]
    o_ref[...] = lax.dot_general(w2_ref[...], u, (((1,), (1,)), ((), ())),
                                 preferred_element_type=jnp.float32) + b2_ref[...]


def _head(s1, s2, ngram_len, word_len, W1, b1, W2, b2):
    B, D = s1.shape
    OUTV, OUTD = W2.shape
    BM = 512
    grid = (B // BM,)
    nl = ngram_len.reshape(B, 1)
    wl = word_len.reshape(B, 1)
    yt = pl.pallas_call(
        _head_body,
        grid=grid,
        in_specs=[
            pl.BlockSpec((BM, D), lambda i: (i, 0)),
            pl.BlockSpec((BM, D), lambda i: (i, 0)),
            pl.BlockSpec((BM, 1), lambda i: (i, 0)),
            pl.BlockSpec((BM, 1), lambda i: (i, 0)),
            pl.BlockSpec((OUTD, 2 * D), lambda i: (0, 0)),
            pl.BlockSpec((1, OUTD), lambda i: (0, 0)),
            pl.BlockSpec((OUTV, OUTD), lambda i: (0, 0)),
            pl.BlockSpec((OUTV, 1), lambda i: (0, 0)),
        ],
        out_specs=pl.BlockSpec((OUTV, BM), lambda i: (0, i)),
        out_shape=jax.ShapeDtypeStruct((OUTV, B), jnp.float32),
    )(s1, s2, nl, wl, W1, b1.reshape(1, OUTD), W2, b2.reshape(OUTV, 1))
    return yt.T


def kernel(words, word_len, ngrams, ngram_len, ngram_table, word_table,
           W1, b1, W2, b2):
    B, LW = words.shape
    _, LN = ngrams.shape
    WV, WD = word_table.shape
    NV, ND = ngram_table.shape

    ngrams_t = ngrams.astype(jnp.int32).T
    words_t = words.astype(jnp.int32).T

    s1 = _pooled_sum_kernel(B, LN, NV, ND, 32)(ngram_table, ngrams_t)
    s2 = _pooled_sum_kernel(B, LW, WV, WD, 64)(word_table, words_t)
    return _head(s1, s2, ngram_len, word_len, W1, b1, W2, b2)
